# baseline (speedup 1.0000x reference)
import functools

import jax
import jax.numpy as jnp
from jax import lax
from jax.experimental import pallas as pl
from jax.experimental.pallas import tpu as pltpu
from jax.experimental.pallas import tpu_sc as plsc

B, HN, N1, DIM = 4096, 12, 200, 256
D2 = 2 * DIM
NC, NS = 2, 16
NW = NC * NS
B_PER_W = B // NW
NB = 4
NBLK = B_PER_W // NB
NG = NBLK // 2
NR = NB * HN
A_DATA = NR * N1
A_ZTOP = A_DATA + (HN - 1) * N1 + 24
H_DATA = NR * DIM
O_PITCH = 208


def _issue(pairs, sem):
    for s, d in pairs:
        pltpu.async_copy(s, d, sem)


def _drain(pairs, sem):
    for s, d in pairs:
        pltpu.make_async_copy(s, d, sem).wait()


def _scatter_sc_kernel(bpw, a_rgb, a_tir, idx_h, vex,
                       idx0, argb0, atir0, vex0,
                       idx1, argb1, atir1, vex1,
                       inv_v, si0, si1, so0, so1):
    nblk = bpw // NB
    ng = nblk // 2
    wid = lax.axis_index("s") * NC + lax.axis_index("c")
    lane = lax.iota(jnp.int32, 16)
    zero16f = jnp.zeros((16,), jnp.float32)
    bufs = [(idx0, argb0, atir0, vex0, si0, so0),
            (idx1, argb1, atir1, vex1, si1, so1)]

    def zz(z, _):
        argb0[pl.ds(A_DATA + z * 16, 16)] = zero16f
        atir0[pl.ds(A_DATA + z * 16, 16)] = zero16f
        argb1[pl.ds(A_DATA + z * 16, 16)] = zero16f
        atir1[pl.ds(A_DATA + z * 16, 16)] = zero16f
        return 0
    lax.fori_loop(0, (A_ZTOP - A_DATA) // 16, zz, 0, unroll=4)

    def in_pairs(t, p):
        idx_v, argb_v, atir_v = bufs[p][0], bufs[p][1], bufs[p][2]
        bbase = wid * bpw + t * NB
        rbase = bbase * HN
        pr = [(idx_h.at[pl.ds((bbase + bb) * N1, N1)],
               idx_v.at[pl.ds(bb * 208, N1)]) for bb in range(NB)]
        pr.append((a_rgb.at[pl.ds(rbase * N1, A_DATA)],
                   argb_v.at[pl.ds(0, A_DATA)]))
        pr.append((a_tir.at[pl.ds(rbase * N1, A_DATA)],
                   atir_v.at[pl.ds(0, A_DATA)]))
        return pr

    def out_pairs(t, p):
        rbase = (wid * bpw + t * NB) * HN
        return [(bufs[p][3], vex.at[pl.ds(rbase * D2, NR * D2)])]

    def compute(t, p):
        idx_v, argb_v, atir_v, vex_v = (bufs[p][0], bufs[p][1], bufs[p][2],
                                        bufs[p][3])
        for bb in range(NB):
            sent = A_DATA - bb * HN * N1
            for c in range(16):
                inv_v[pl.ds(c * 16, 16)] = jnp.full((16,), sent, jnp.int32)
            for c in range(13):
                raw = idx_v[pl.ds(bb * 208 + c * 16, 16)]
                if c == 12:
                    raw = jnp.where(lane < 8, raw, 256 + lane)
                _, last_mask = plsc.scan_count(raw)
                plsc.store_scatter(inv_v, [raw], c * 16 + lane,
                                   mask=last_mask)
            cols = [inv_v[pl.ds(c * 16, 16)] for c in range(16)]

            @plsc.parallel_loop(0, HN, unroll=2)
            def _(r):
                aoff = (bb * HN + r) * N1
                voff = (bb * HN + r) * D2
                for c in range(16):
                    col = cols[c] + aoff
                    vex_v[pl.ds(voff + c * 16, 16)] = (
                        plsc.load_gather(argb_v, [col]))
                    vex_v[pl.ds(voff + DIM + c * 16, 16)] = (
                        plsc.load_gather(atir_v, [col]))

    _issue(in_pairs(0, 0), si0)

    def group(g, _):
        t0 = 2 * g
        _issue(in_pairs(t0 + 1, 1), si1)
        _drain(in_pairs(t0, 0), si0)

        @pl.when(g > 0)
        def _():
            _drain(out_pairs(t0 - 2, 0), so0)
        compute(t0, 0)
        _issue(out_pairs(t0, 0), so0)

        @pl.when(g + 1 < ng)
        def _():
            _issue(in_pairs(t0 + 2, 0), si0)
        _drain(in_pairs(t0 + 1, 1), si1)

        @pl.when(g > 0)
        def _():
            _drain(out_pairs(t0 - 1, 1), so1)
        compute(t0 + 1, 1)
        _issue(out_pairs(t0 + 1, 1), so1)
        return 0

    lax.fori_loop(0, ng, group, 0)
    _drain(out_pairs(nblk - 2, 0), so0)
    _drain(out_pairs(nblk - 1, 1), so1)


def _mlp_tc_kernel(x_ref, w1g_ref, gv_ref, bw_ref, w2_ref, b2_ref, o_ref):
    x = x_ref[...].reshape(-1, D2)
    mu = jnp.mean(x, axis=1, keepdims=True)
    msq = jnp.mean(x * x, axis=1, keepdims=True)
    rstd = lax.rsqrt(msq - mu * mu + 1e-5)
    u = jnp.dot(x.astype(jnp.bfloat16), w1g_ref[...],
                preferred_element_type=jnp.float32)
    x1 = u * rstd - (mu * rstd) * gv_ref[...] + bw_ref[...]
    h1 = jnp.maximum(x1, 0.0).astype(jnp.bfloat16)
    h2 = jnp.dot(h1, w2_ref[...], preferred_element_type=jnp.float32)
    o_ref[...] = jax.nn.sigmoid(h2 + b2_ref[...]).reshape(-1)


def _gather_sc_kernel(bpw, hmat, idx_h, out,
                      idx0, h0, out0, idx1, h1, out1,
                      si0, si1, so0, so1):
    nblk = bpw // NB
    ng = nblk // 2
    wid = lax.axis_index("s") * NC + lax.axis_index("c")
    lane = lax.iota(jnp.int32, 16)
    bufs = [(idx0, h0, out0, si0, so0), (idx1, h1, out1, si1, so1)]

    def in_pairs(t, p):
        idx_v, h_v = bufs[p][0], bufs[p][1]
        bbase = wid * bpw + t * NB
        pr = [(idx_h.at[pl.ds((bbase + bb) * N1, N1)],
               idx_v.at[pl.ds(bb * 208, N1)]) for bb in range(NB)]
        pr.append((hmat.at[pl.ds(bbase * HN * DIM, H_DATA)], h_v))
        return pr

    def out_pairs(t, p):
        out_v = bufs[p][2]
        rbase = (wid * bpw + t * NB) * HN
        return [(out_v.at[pl.ds(r * O_PITCH, N1)],
                 out.at[pl.ds((rbase + r) * N1, N1)]) for r in range(NR)]

    def compute(t, p):
        idx_v, h_v, out_v = bufs[p][0], bufs[p][1], bufs[p][2]
        for bb in range(NB):
            chunks = []
            for c in range(13):
                raw = idx_v[pl.ds(bb * 208 + c * 16, 16)]
                if c == 12:
                    raw = jnp.where(lane < 8, raw, 0)
                chunks.append(raw)

            @plsc.parallel_loop(0, HN, unroll=2)
            def _(r):
                hoff = (bb * HN + r) * DIM
                ooff = (bb * HN + r) * O_PITCH
                for c in range(13):
                    out_v[pl.ds(ooff + c * 16, 16)] = (
                        plsc.load_gather(h_v, [chunks[c] + hoff]))

    _issue(in_pairs(0, 0), si0)

    def group(g, _):
        t0 = 2 * g
        _issue(in_pairs(t0 + 1, 1), si1)
        _drain(in_pairs(t0, 0), si0)

        @pl.when(g > 0)
        def _():
            _drain(out_pairs(t0 - 2, 0), so0)
        compute(t0, 0)
        _issue(out_pairs(t0, 0), so0)

        @pl.when(g + 1 < ng)
        def _():
            _issue(in_pairs(t0 + 2, 0), si0)
        _drain(in_pairs(t0 + 1, 1), si1)

        @pl.when(g > 0)
        def _():
            _drain(out_pairs(t0 - 1, 1), so1)
        compute(t0 + 1, 1)
        _issue(out_pairs(t0 + 1, 1), so1)
        return 0

    lax.fori_loop(0, ng, group, 0)
    _drain(out_pairs(nblk - 2, 0), so0)
    _drain(out_pairs(nblk - 1, 1), so1)


NCHUNK = 2
CB = B // NCHUNK
CBPW = CB // NW


def _run_chunk(a_rgb3, a_tir3, idx2, w1g, gv, bw, w2b, b2r):
    a_rgb = a_rgb3.reshape(CB * HN * N1)
    a_tir = a_tir3.reshape(CB * HN * N1)
    idx_f = idx2.reshape(CB * N1)

    mesh = plsc.VectorSubcoreMesh(core_axis_name="c", subcore_axis_name="s")
    sc_params = pltpu.CompilerParams(needs_layout_passes=False)
    scatter = pl.kernel(
        functools.partial(_scatter_sc_kernel, CBPW),
        mesh=mesh,
        compiler_params=sc_params,
        out_type=jax.ShapeDtypeStruct((CB * HN * D2,), jnp.float32),
        scratch_types=[
            pltpu.VMEM((NB * 208,), jnp.int32),
            pltpu.VMEM((A_ZTOP,), jnp.float32),
            pltpu.VMEM((A_ZTOP,), jnp.float32),
            pltpu.VMEM((NR * D2,), jnp.float32),
            pltpu.VMEM((NB * 208,), jnp.int32),
            pltpu.VMEM((A_ZTOP,), jnp.float32),
            pltpu.VMEM((A_ZTOP,), jnp.float32),
            pltpu.VMEM((NR * D2,), jnp.float32),
            pltpu.VMEM((272,), jnp.int32),
            pltpu.SemaphoreType.DMA,
            pltpu.SemaphoreType.DMA,
            pltpu.SemaphoreType.DMA,
            pltpu.SemaphoreType.DMA,
        ],
    )
    vex = scatter(a_rgb, a_tir, idx_f)

    nrows = CB * HN
    blk = 512
    hmat = pl.pallas_call(
        _mlp_tc_kernel,
        grid=(nrows // blk,),
        in_specs=[
            pl.BlockSpec((blk * D2,), lambda i: (i,)),
            pl.BlockSpec((D2, DIM), lambda i: (0, 0)),
            pl.BlockSpec((1, DIM), lambda i: (0, 0)),
            pl.BlockSpec((1, DIM), lambda i: (0, 0)),
            pl.BlockSpec((DIM, DIM), lambda i: (0, 0)),
            pl.BlockSpec((1, DIM), lambda i: (0, 0)),
        ],
        out_specs=pl.BlockSpec((blk * DIM,), lambda i: (i,)),
        out_shape=jax.ShapeDtypeStruct((nrows * DIM,), jnp.float32),
    )(vex, w1g, gv, bw, w2b, b2r)

    gather = pl.kernel(
        functools.partial(_gather_sc_kernel, CBPW),
        mesh=mesh,
        compiler_params=sc_params,
        out_type=jax.ShapeDtypeStruct((CB * HN * N1,), jnp.float32),
        scratch_types=[
            pltpu.VMEM((NB * 208,), jnp.int32),
            pltpu.VMEM((H_DATA,), jnp.float32),
            pltpu.VMEM((NR * O_PITCH + 16,), jnp.float32),
            pltpu.VMEM((NB * 208,), jnp.int32),
            pltpu.VMEM((H_DATA,), jnp.float32),
            pltpu.VMEM((NR * O_PITCH + 16,), jnp.float32),
            pltpu.SemaphoreType.DMA,
            pltpu.SemaphoreType.DMA,
            pltpu.SemaphoreType.DMA,
            pltpu.SemaphoreType.DMA,
        ],
    )
    out = gather(hmat, idx_f)
    return out.reshape(CB, HN, N1)


@jax.jit
def kernel(attn_rgb_weight, attn_tir_weight, global_index_s, ln_g, ln_b,
           W1, b1, W2, b2):
    w1g = (ln_g[:, None] * W1).astype(jnp.bfloat16)
    gv = (ln_g @ W1).reshape(1, DIM)
    bw = (ln_b @ W1 + b1).reshape(1, DIM)
    w2b = W2.astype(jnp.bfloat16)
    b2r = b2.reshape(1, DIM)
    outs = []
    for c in range(NCHUNK):
        sl = slice(c * CB, (c + 1) * CB)
        outs.append(_run_chunk(attn_rgb_weight[sl], attn_tir_weight[sl],
                               global_index_s[sl], w1g, gv, bw, w2b, b2r))
    return jnp.concatenate(outs, axis=0)


# speedup vs baseline: 1.0327x; 1.0327x over previous
import jax
import jax.numpy as jnp
from jax import lax
from jax.experimental import pallas as pl
from jax.experimental.pallas import tpu as pltpu
from jax.experimental.pallas import tpu_sc as plsc

B, HN, N1, DIM = 4096, 12, 200, 256
D2 = 2 * DIM
NC, NS = 2, 16
NW = NC * NS
B_PER_W = B // NW
NB = 4
NBLK = B_PER_W // NB
NG = NBLK // 2
NR = NB * HN
A_DATA = NR * N1
A_ZTOP = A_DATA + (HN - 1) * N1 + 24
H_DATA = NR * DIM
O_PITCH = 200


def _issue(pairs, sem):
    for s, d in pairs:
        pltpu.async_copy(s, d, sem)


def _drain(pairs, sem):
    for s, d in pairs:
        pltpu.make_async_copy(s, d, sem).wait()


def _scatter_sc_kernel(a_rgb, a_tir, idx_h, vex,
                       idx0, argb0, atir0, vex0,
                       idx1, argb1, atir1, vex1,
                       inv_v, si0, si1, so0, so1):
    wid = lax.axis_index("s") * NC + lax.axis_index("c")
    lane = lax.iota(jnp.int32, 16)
    zero16f = jnp.zeros((16,), jnp.float32)
    bufs = [(idx0, argb0, atir0, vex0, si0, so0),
            (idx1, argb1, atir1, vex1, si1, so1)]

    def zz(z, _):
        argb0[pl.ds(A_DATA + z * 16, 16)] = zero16f
        atir0[pl.ds(A_DATA + z * 16, 16)] = zero16f
        argb1[pl.ds(A_DATA + z * 16, 16)] = zero16f
        atir1[pl.ds(A_DATA + z * 16, 16)] = zero16f
        return 0
    lax.fori_loop(0, (A_ZTOP - A_DATA) // 16, zz, 0, unroll=4)

    def in_pairs(t, p):
        idx_v, argb_v, atir_v = bufs[p][0], bufs[p][1], bufs[p][2]
        bbase = wid * B_PER_W + t * NB
        rbase = bbase * HN
        pr = [(idx_h.at[pl.ds((bbase + bb) * N1, N1)],
               idx_v.at[pl.ds(bb * 208, N1)]) for bb in range(NB)]
        pr.append((a_rgb.at[pl.ds(rbase * N1, A_DATA)],
                   argb_v.at[pl.ds(0, A_DATA)]))
        pr.append((a_tir.at[pl.ds(rbase * N1, A_DATA)],
                   atir_v.at[pl.ds(0, A_DATA)]))
        return pr

    def out_pairs(t, p):
        rbase = (wid * B_PER_W + t * NB) * HN
        return [(bufs[p][3], vex.at[pl.ds(rbase * D2, NR * D2)])]

    def compute(t, p):
        idx_v, argb_v, atir_v, vex_v = (bufs[p][0], bufs[p][1], bufs[p][2],
                                        bufs[p][3])
        for bb in range(NB):
            sent = A_DATA - bb * HN * N1
            for c in range(16):
                inv_v[pl.ds(c * 16, 16)] = jnp.full((16,), sent, jnp.int32)
            for c in range(13):
                raw = idx_v[pl.ds(bb * 208 + c * 16, 16)]
                if c == 12:
                    raw = jnp.where(lane < 8, raw, 256 + lane)
                _, last_mask = plsc.scan_count(raw)
                plsc.store_scatter(inv_v, [raw], c * 16 + lane,
                                   mask=last_mask)
            cols = [inv_v[pl.ds(c * 16, 16)] for c in range(16)]

            @plsc.parallel_loop(0, HN, unroll=3)
            def _(r):
                aoff = (bb * HN + r) * N1
                voff = (bb * HN + r) * D2
                for c in range(16):
                    col = cols[c] + aoff
                    vex_v[pl.ds(voff + c * 16, 16)] = (
                        plsc.load_gather(argb_v, [col]))
                    vex_v[pl.ds(voff + DIM + c * 16, 16)] = (
                        plsc.load_gather(atir_v, [col]))

    _issue(in_pairs(0, 0), si0)

    def group(g, _):
        t0 = 2 * g
        _issue(in_pairs(t0 + 1, 1), si1)
        _drain(in_pairs(t0, 0), si0)

        @pl.when(g > 0)
        def _():
            _drain(out_pairs(t0 - 2, 0), so0)
        compute(t0, 0)
        _issue(out_pairs(t0, 0), so0)

        @pl.when(g + 1 < NG)
        def _():
            _issue(in_pairs(t0 + 2, 0), si0)
        _drain(in_pairs(t0 + 1, 1), si1)

        @pl.when(g > 0)
        def _():
            _drain(out_pairs(t0 - 1, 1), so1)
        compute(t0 + 1, 1)
        _issue(out_pairs(t0 + 1, 1), so1)
        return 0

    lax.fori_loop(0, NG, group, 0)
    _drain(out_pairs(NBLK - 2, 0), so0)
    _drain(out_pairs(NBLK - 1, 1), so1)


def _mlp_tc_kernel(x_ref, w1g_ref, gv_ref, bw_ref, w2_ref, b2_ref, o_ref):
    x = x_ref[...].reshape(-1, D2)
    mu = jnp.mean(x, axis=1, keepdims=True)
    msq = jnp.mean(x * x, axis=1, keepdims=True)
    rstd = lax.rsqrt(msq - mu * mu + 1e-5)
    u = jnp.dot(x.astype(jnp.bfloat16), w1g_ref[...],
                preferred_element_type=jnp.float32)
    x1 = u * rstd - (mu * rstd) * gv_ref[...] + bw_ref[...]
    h1 = jnp.maximum(x1, 0.0).astype(jnp.bfloat16)
    h2 = jnp.dot(h1, w2_ref[...], preferred_element_type=jnp.float32)
    o_ref[...] = jax.nn.sigmoid(h2 + b2_ref[...]).reshape(-1)


def _gather_sc_kernel(hmat, idx_h, out,
                      idx0, h0, out0, idx1, h1, out1,
                      si0, si1, so0, so1):
    wid = lax.axis_index("s") * NC + lax.axis_index("c")
    lane = lax.iota(jnp.int32, 16)
    bufs = [(idx0, h0, out0, si0, so0), (idx1, h1, out1, si1, so1)]

    def in_pairs(t, p):
        idx_v, h_v = bufs[p][0], bufs[p][1]
        bbase = wid * B_PER_W + t * NB
        pr = [(idx_h.at[pl.ds((bbase + bb) * N1, N1)],
               idx_v.at[pl.ds(bb * 208, N1)]) for bb in range(NB)]
        pr.append((hmat.at[pl.ds(bbase * HN * DIM, H_DATA)], h_v))
        return pr

    def out_pairs(t, p):
        out_v = bufs[p][2]
        rbase = (wid * B_PER_W + t * NB) * HN
        return [(out_v.at[pl.ds(0, NR * N1)],
                 out.at[pl.ds(rbase * N1, NR * N1)])]

    def compute(t, p):
        idx_v, h_v, out_v = bufs[p][0], bufs[p][1], bufs[p][2]
        for bb in range(NB):
            chunks = []
            for c in range(13):
                raw = idx_v[pl.ds(bb * 208 + c * 16, 16)]
                if c == 12:
                    raw = jnp.where(lane < 8, raw, 0)
                chunks.append(raw)

            @plsc.parallel_loop(0, HN, unroll=2)
            def _(r):
                hoff = (bb * HN + r) * DIM
                ooff = (bb * HN + r) * O_PITCH
                for c in range(12):
                    out_v[pl.ds(ooff + c * 16, 16)] = (
                        plsc.load_gather(h_v, [chunks[c] + hoff]))
                g12 = plsc.load_gather(h_v, [chunks[12] + hoff])
                plsc.store_scatter(out_v, [ooff + 192 + lane], g12,
                                   mask=lane < 8)

    _issue(in_pairs(0, 0), si0)

    def group(g, _):
        t0 = 2 * g
        _issue(in_pairs(t0 + 1, 1), si1)
        _drain(in_pairs(t0, 0), si0)

        @pl.when(g > 0)
        def _():
            _drain(out_pairs(t0 - 2, 0), so0)
        compute(t0, 0)
        _issue(out_pairs(t0, 0), so0)

        @pl.when(g + 1 < NG)
        def _():
            _issue(in_pairs(t0 + 2, 0), si0)
        _drain(in_pairs(t0 + 1, 1), si1)

        @pl.when(g > 0)
        def _():
            _drain(out_pairs(t0 - 1, 1), so1)
        compute(t0 + 1, 1)
        _issue(out_pairs(t0 + 1, 1), so1)
        return 0

    lax.fori_loop(0, NG, group, 0)
    _drain(out_pairs(NBLK - 2, 0), so0)
    _drain(out_pairs(NBLK - 1, 1), so1)


@jax.jit
def kernel(attn_rgb_weight, attn_tir_weight, global_index_s, ln_g, ln_b,
           W1, b1, W2, b2):
    a_rgb = attn_rgb_weight.reshape(B * HN * N1)
    a_tir = attn_tir_weight.reshape(B * HN * N1)
    idx_f = global_index_s.reshape(B * N1)

    mesh = plsc.VectorSubcoreMesh(core_axis_name="c", subcore_axis_name="s")
    sc_params = pltpu.CompilerParams(needs_layout_passes=False)
    scatter = pl.kernel(
        _scatter_sc_kernel,
        mesh=mesh,
        compiler_params=sc_params,
        out_type=jax.ShapeDtypeStruct((B * HN * D2,), jnp.float32),
        scratch_types=[
            pltpu.VMEM((NB * 208,), jnp.int32),
            pltpu.VMEM((A_ZTOP,), jnp.float32),
            pltpu.VMEM((A_ZTOP,), jnp.float32),
            pltpu.VMEM((NR * D2,), jnp.float32),
            pltpu.VMEM((NB * 208,), jnp.int32),
            pltpu.VMEM((A_ZTOP,), jnp.float32),
            pltpu.VMEM((A_ZTOP,), jnp.float32),
            pltpu.VMEM((NR * D2,), jnp.float32),
            pltpu.VMEM((272,), jnp.int32),
            pltpu.SemaphoreType.DMA,
            pltpu.SemaphoreType.DMA,
            pltpu.SemaphoreType.DMA,
            pltpu.SemaphoreType.DMA,
        ],
    )
    vex = scatter(a_rgb, a_tir, idx_f)

    nrows = B * HN
    blk = 512
    w1g = (ln_g[:, None] * W1).astype(jnp.bfloat16)
    gv = (ln_g @ W1).reshape(1, DIM)
    bw = (ln_b @ W1 + b1).reshape(1, DIM)
    hmat = pl.pallas_call(
        _mlp_tc_kernel,
        grid=(nrows // blk,),
        in_specs=[
            pl.BlockSpec((blk * D2,), lambda i: (i,)),
            pl.BlockSpec((D2, DIM), lambda i: (0, 0)),
            pl.BlockSpec((1, DIM), lambda i: (0, 0)),
            pl.BlockSpec((1, DIM), lambda i: (0, 0)),
            pl.BlockSpec((DIM, DIM), lambda i: (0, 0)),
            pl.BlockSpec((1, DIM), lambda i: (0, 0)),
        ],
        out_specs=pl.BlockSpec((blk * DIM,), lambda i: (i,)),
        out_shape=jax.ShapeDtypeStruct((nrows * DIM,), jnp.float32),
    )(vex, w1g, gv, bw, W2.astype(jnp.bfloat16), b2.reshape(1, DIM))

    gather = pl.kernel(
        _gather_sc_kernel,
        mesh=mesh,
        compiler_params=sc_params,
        out_type=jax.ShapeDtypeStruct((B * HN * N1,), jnp.float32),
        scratch_types=[
            pltpu.VMEM((NB * 208,), jnp.int32),
            pltpu.VMEM((H_DATA,), jnp.float32),
            pltpu.VMEM((NR * O_PITCH + 16,), jnp.float32),
            pltpu.VMEM((NB * 208,), jnp.int32),
            pltpu.VMEM((H_DATA,), jnp.float32),
            pltpu.VMEM((NR * O_PITCH + 16,), jnp.float32),
            pltpu.SemaphoreType.DMA,
            pltpu.SemaphoreType.DMA,
            pltpu.SemaphoreType.DMA,
            pltpu.SemaphoreType.DMA,
        ],
    )
    out = gather(hmat, idx_f)
    return out.reshape(B, HN, N1)


# speedup vs baseline: 1.0358x; 1.0030x over previous
import jax
import jax.numpy as jnp
from jax import lax
from jax.experimental import pallas as pl
from jax.experimental.pallas import tpu as pltpu
from jax.experimental.pallas import tpu_sc as plsc

B, HN, N1, DIM = 4096, 12, 200, 256
D2 = 2 * DIM
NC, NS = 2, 16
NW = NC * NS
B_PER_W = B // NW
NB = 4
NBLK = B_PER_W // NB
NG = NBLK // 2
NR = NB * HN
A_DATA = NR * N1
A_ZTOP = A_DATA + (HN - 1) * N1 + 24
H_DATA = NR * DIM
O_PITCH = 200


def _issue(pairs, sem):
    for s, d in pairs:
        pltpu.async_copy(s, d, sem)


def _drain(pairs, sem):
    for s, d in pairs:
        pltpu.make_async_copy(s, d, sem).wait()


def _scatter_sc_kernel(a_rgb, a_tir, idx_h, vex,
                       idx0, argb0, atir0, vex0,
                       idx1, argb1, atir1, vex1,
                       inv_v, si0, si1, so0, so1):
    wid = lax.axis_index("s") * NC + lax.axis_index("c")
    lane = lax.iota(jnp.int32, 16)
    zero16f = jnp.zeros((16,), jnp.float32)
    bufs = [(idx0, argb0, atir0, vex0, si0, so0),
            (idx1, argb1, atir1, vex1, si1, so1)]

    def zz(z, _):
        argb0[pl.ds(A_DATA + z * 16, 16)] = zero16f
        atir0[pl.ds(A_DATA + z * 16, 16)] = zero16f
        argb1[pl.ds(A_DATA + z * 16, 16)] = zero16f
        atir1[pl.ds(A_DATA + z * 16, 16)] = zero16f
        return 0
    lax.fori_loop(0, (A_ZTOP - A_DATA) // 16, zz, 0, unroll=4)

    def in_pairs(t, p):
        idx_v, argb_v, atir_v = bufs[p][0], bufs[p][1], bufs[p][2]
        bbase = wid * B_PER_W + t * NB
        rbase = bbase * HN
        pr = [(idx_h.at[pl.ds((bbase + bb) * N1, N1)],
               idx_v.at[pl.ds(bb * 208, N1)]) for bb in range(NB)]
        pr.append((a_rgb.at[pl.ds(rbase * N1, A_DATA)],
                   argb_v.at[pl.ds(0, A_DATA)]))
        pr.append((a_tir.at[pl.ds(rbase * N1, A_DATA)],
                   atir_v.at[pl.ds(0, A_DATA)]))
        return pr

    def out_pairs(t, p):
        rbase = (wid * B_PER_W + t * NB) * HN
        return [(bufs[p][3], vex.at[pl.ds(rbase * D2, NR * D2)])]

    def compute(t, p):
        idx_v, argb_v, atir_v, vex_v = (bufs[p][0], bufs[p][1], bufs[p][2],
                                        bufs[p][3])
        for bb in range(NB):
            sent = A_DATA - bb * HN * N1
            for c in range(16):
                inv_v[pl.ds(c * 16, 16)] = jnp.full((16,), sent, jnp.int32)
            for c in range(13):
                raw = idx_v[pl.ds(bb * 208 + c * 16, 16)]
                if c == 12:
                    raw = jnp.where(lane < 8, raw, 256 + lane)
                _, last_mask = plsc.scan_count(raw)
                plsc.store_scatter(inv_v, [raw], c * 16 + lane,
                                   mask=last_mask)
            cols = [inv_v[pl.ds(c * 16, 16)] for c in range(16)]

            @plsc.parallel_loop(0, HN, unroll=3)
            def _(r):
                aoff = (bb * HN + r) * N1
                voff = (bb * HN + r) * D2
                for c in range(16):
                    col = cols[c] + aoff
                    vex_v[pl.ds(voff + c * 16, 16)] = (
                        plsc.load_gather(argb_v, [col]))
                    vex_v[pl.ds(voff + DIM + c * 16, 16)] = (
                        plsc.load_gather(atir_v, [col]))

    _issue(in_pairs(0, 0), si0)

    def group(g, _):
        t0 = 2 * g
        _issue(in_pairs(t0 + 1, 1), si1)
        _drain(in_pairs(t0, 0), si0)

        @pl.when(g > 0)
        def _():
            _drain(out_pairs(t0 - 2, 0), so0)
        compute(t0, 0)
        _issue(out_pairs(t0, 0), so0)

        @pl.when(g + 1 < NG)
        def _():
            _issue(in_pairs(t0 + 2, 0), si0)
        _drain(in_pairs(t0 + 1, 1), si1)

        @pl.when(g > 0)
        def _():
            _drain(out_pairs(t0 - 1, 1), so1)
        compute(t0 + 1, 1)
        _issue(out_pairs(t0 + 1, 1), so1)
        return 0

    lax.fori_loop(0, NG, group, 0)
    _drain(out_pairs(NBLK - 2, 0), so0)
    _drain(out_pairs(NBLK - 1, 1), so1)


def _mlp_tc_kernel(x_ref, w1g_ref, gv_ref, bw_ref, w2_ref, b2_ref, o_ref):
    x = x_ref[...].reshape(-1, D2)
    mu = jnp.mean(x, axis=1, keepdims=True)
    msq = jnp.mean(x * x, axis=1, keepdims=True)
    rstd = lax.rsqrt(msq - mu * mu + 1e-5)
    u = jnp.dot(x.astype(jnp.bfloat16), w1g_ref[...],
                preferred_element_type=jnp.float32)
    x1 = u * rstd - (mu * rstd) * gv_ref[...] + bw_ref[...]
    h1 = jnp.maximum(x1, 0.0).astype(jnp.bfloat16)
    h2 = jnp.dot(h1, w2_ref[...], preferred_element_type=jnp.float32)
    o_ref[...] = jax.nn.sigmoid(h2 + b2_ref[...]).reshape(-1)


def _gather_sc_kernel(hmat, idx_h, out,
                      idx0, h0, out0, idx1, h1, out1,
                      si0, si1, so0, so1):
    wid = lax.axis_index("s") * NC + lax.axis_index("c")
    lane = lax.iota(jnp.int32, 16)
    bufs = [(idx0, h0, out0, si0, so0), (idx1, h1, out1, si1, so1)]

    def in_pairs(t, p):
        idx_v, h_v = bufs[p][0], bufs[p][1]
        bbase = wid * B_PER_W + t * NB
        pr = [(idx_h.at[pl.ds((bbase + bb) * N1, N1)],
               idx_v.at[pl.ds(bb * 208, N1)]) for bb in range(NB)]
        pr.append((hmat.at[pl.ds(bbase * HN * DIM, H_DATA)], h_v))
        return pr

    def out_pairs(t, p):
        out_v = bufs[p][2]
        rbase = (wid * B_PER_W + t * NB) * HN
        return [(out_v.at[pl.ds(0, NR * N1)],
                 out.at[pl.ds(rbase * N1, NR * N1)])]

    def compute(t, p):
        idx_v, h_v, out_v = bufs[p][0], bufs[p][1], bufs[p][2]
        for bb in range(NB):
            chunks = []
            for c in range(13):
                raw = idx_v[pl.ds(bb * 208 + c * 16, 16)]
                if c == 12:
                    raw = jnp.where(lane < 8, raw, 0)
                chunks.append(raw)

            @plsc.parallel_loop(0, HN, unroll=2)
            def _(r):
                hoff = (bb * HN + r) * DIM
                ooff = (bb * HN + r) * O_PITCH
                for c in range(12):
                    out_v[pl.ds(ooff + c * 16, 16)] = (
                        plsc.load_gather(h_v, [chunks[c] + hoff]))
                g12 = plsc.load_gather(h_v, [chunks[12] + hoff])
                plsc.store_scatter(out_v, [ooff + 192 + lane], g12,
                                   mask=lane < 8)

    _issue(in_pairs(0, 0), si0)

    def group(g, _):
        t0 = 2 * g
        _issue(in_pairs(t0 + 1, 1), si1)
        _drain(in_pairs(t0, 0), si0)

        @pl.when(g > 0)
        def _():
            _drain(out_pairs(t0 - 2, 0), so0)
        compute(t0, 0)
        _issue(out_pairs(t0, 0), so0)

        @pl.when(g + 1 < NG)
        def _():
            _issue(in_pairs(t0 + 2, 0), si0)
        _drain(in_pairs(t0 + 1, 1), si1)

        @pl.when(g > 0)
        def _():
            _drain(out_pairs(t0 - 1, 1), so1)
        compute(t0 + 1, 1)
        _issue(out_pairs(t0 + 1, 1), so1)
        return 0

    lax.fori_loop(0, NG, group, 0)
    _drain(out_pairs(NBLK - 2, 0), so0)
    _drain(out_pairs(NBLK - 1, 1), so1)


def _to_flat(x, n):
    y = jax.lax.optimization_barrier(x.reshape(n // 128, 128))
    return y.reshape(n)


@jax.jit
def kernel(attn_rgb_weight, attn_tir_weight, global_index_s, ln_g, ln_b,
           W1, b1, W2, b2):
    a_rgb = _to_flat(attn_rgb_weight, B * HN * N1)
    a_tir = _to_flat(attn_tir_weight, B * HN * N1)
    idx_f = global_index_s.reshape(B * N1)

    mesh = plsc.VectorSubcoreMesh(core_axis_name="c", subcore_axis_name="s")
    sc_params = pltpu.CompilerParams(needs_layout_passes=False)
    scatter = pl.kernel(
        _scatter_sc_kernel,
        mesh=mesh,
        compiler_params=sc_params,
        out_type=jax.ShapeDtypeStruct((B * HN * D2,), jnp.float32),
        scratch_types=[
            pltpu.VMEM((NB * 208,), jnp.int32),
            pltpu.VMEM((A_ZTOP,), jnp.float32),
            pltpu.VMEM((A_ZTOP,), jnp.float32),
            pltpu.VMEM((NR * D2,), jnp.float32),
            pltpu.VMEM((NB * 208,), jnp.int32),
            pltpu.VMEM((A_ZTOP,), jnp.float32),
            pltpu.VMEM((A_ZTOP,), jnp.float32),
            pltpu.VMEM((NR * D2,), jnp.float32),
            pltpu.VMEM((272,), jnp.int32),
            pltpu.SemaphoreType.DMA,
            pltpu.SemaphoreType.DMA,
            pltpu.SemaphoreType.DMA,
            pltpu.SemaphoreType.DMA,
        ],
    )
    vex = scatter(a_rgb, a_tir, idx_f)

    nrows = B * HN
    blk = 512
    w1g = (ln_g[:, None] * W1).astype(jnp.bfloat16)
    gv = (ln_g @ W1).reshape(1, DIM)
    bw = (ln_b @ W1 + b1).reshape(1, DIM)
    hmat = pl.pallas_call(
        _mlp_tc_kernel,
        grid=(nrows // blk,),
        in_specs=[
            pl.BlockSpec((blk * D2,), lambda i: (i,)),
            pl.BlockSpec((D2, DIM), lambda i: (0, 0)),
            pl.BlockSpec((1, DIM), lambda i: (0, 0)),
            pl.BlockSpec((1, DIM), lambda i: (0, 0)),
            pl.BlockSpec((DIM, DIM), lambda i: (0, 0)),
            pl.BlockSpec((1, DIM), lambda i: (0, 0)),
        ],
        out_specs=pl.BlockSpec((blk * DIM,), lambda i: (i,)),
        out_shape=jax.ShapeDtypeStruct((nrows * DIM,), jnp.float32),
    )(vex, w1g, gv, bw, W2.astype(jnp.bfloat16), b2.reshape(1, DIM))

    gather = pl.kernel(
        _gather_sc_kernel,
        mesh=mesh,
        compiler_params=sc_params,
        out_type=jax.ShapeDtypeStruct((B * HN * N1,), jnp.float32),
        scratch_types=[
            pltpu.VMEM((NB * 208,), jnp.int32),
            pltpu.VMEM((H_DATA,), jnp.float32),
            pltpu.VMEM((NR * O_PITCH + 16,), jnp.float32),
            pltpu.VMEM((NB * 208,), jnp.int32),
            pltpu.VMEM((H_DATA,), jnp.float32),
            pltpu.VMEM((NR * O_PITCH + 16,), jnp.float32),
            pltpu.SemaphoreType.DMA,
            pltpu.SemaphoreType.DMA,
            pltpu.SemaphoreType.DMA,
            pltpu.SemaphoreType.DMA,
        ],
    )
    out = gather(hmat, idx_f)
    out2 = jax.lax.optimization_barrier(out.reshape(B * HN * N1 // 128, 128))
    return out2.reshape(B, HN, N1)
